# baseline (speedup 1.0000x reference)
import functools

import jax
import jax.numpy as jnp
from jax import lax
from jax.experimental import pallas as pl
from jax.experimental.pallas import tpu as pltpu
from jax.experimental.pallas import tpu_sc as plsc

N = 100000
E = 1600000
H = 32
C = 10
G = 32

NC = 2
NS = 16
LANES = 16

R = 2048
NP = 49 * R
NHALF = NP // 2
ROWS_PER_TILE = NHALF // NS
NTRASH = 1024
TBL_ROWS = NHALF + NTRASH
ZSTRIPE = TBL_ROWS // NS

SUB = 4
GRP = SUB * 128
GPT = 196
EPT = GPT * GRP
EPAD = NS * EPT
NGRP = NS * GPT


def _fill_rows(rows, value, n):
    fill = jnp.full((LANES,), value, jnp.float32)

    def fill_body(i, _):
        rows[i, pl.ds(0, LANES)] = fill
        rows[i, pl.ds(LANES, LANES)] = fill
        return _

    lax.fori_loop(0, n, fill_body, None)


def _redirect(idx, off, nh_u):
    for j in range(SUB):
        for l in range(8):
            v = idx[SUB + j, pl.ds(l * LANES, LANES)]
            u = lax.bitcast_convert_type(v - off, jnp.uint32)
            trash = jnp.uint32(NHALF) + (u & jnp.uint32(NTRASH - 1))
            u = jnp.where(u < nh_u, u, trash)
            idx[SUB + j, pl.ds(l * LANES, LANES)] = lax.bitcast_convert_type(
                u, jnp.int32)


def _agg_body(in_hbm, eidx_hbm, out_hbm, idxa, idxb, rows, table,
              sem_ia, sem_ib, sem_g, sem_s, *, gather: bool):
    c = lax.axis_index("c")
    s = lax.axis_index("s")
    nrows = SUB * 128

    _fill_rows(rows, 0.0, nrows)
    zb = s * ZSTRIPE
    nfull = ZSTRIPE // nrows

    def zero_body(i, _):
        pltpu.sync_copy(rows, table.at[pl.ds(zb + i * nrows, nrows)])
        return _

    lax.fori_loop(0, nfull, zero_body, None)
    rem = ZSTRIPE - nfull * nrows
    pltpu.sync_copy(rows.at[pl.ds(0, rem)],
                    table.at[pl.ds(zb + nfull * nrows, rem)])
    if not gather:
        _fill_rows(rows, 1.0, nrows)

    plsc.subcore_barrier()

    nh_u = jnp.uint32(NHALF)
    off = (c * NHALF).astype(jnp.int32)
    base = s * GPT

    def slot(j):
        return rows.at[pl.ds(j * 128, 128)]

    def drain_scatters(idx):
        for j in range(SUB):
            pltpu.make_async_copy(slot(j), table.at[idx.at[SUB + j]],
                                  sem_s).wait()

    pltpu.async_copy(eidx_hbm.at[base], idxa, sem_ia)

    def body(i, _):
        g = base + 2 * i

        pltpu.make_async_copy(eidx_hbm.at[0], idxa, sem_ia).wait()

        @pl.when(i > 0)
        def _():
            drain_scatters(idxb)
        if gather:
            cps = [pltpu.async_copy(in_hbm.at[idxa.at[j]], slot(j), sem_g)
                   for j in range(SUB)]
        pltpu.async_copy(eidx_hbm.at[g + 1], idxb, sem_ib)
        _redirect(idxa, off, nh_u)
        if gather:
            for cp in cps:
                cp.wait()
        for j in range(SUB):
            pltpu.async_copy(slot(j), table.at[idxa.at[SUB + j]], sem_s,
                             add=True)

        pltpu.make_async_copy(eidx_hbm.at[0], idxb, sem_ib).wait()
        drain_scatters(idxa)
        if gather:
            cps2 = [pltpu.async_copy(in_hbm.at[idxb.at[j]], slot(j), sem_g)
                    for j in range(SUB)]
        pltpu.async_copy(eidx_hbm.at[g + 2], idxa, sem_ia)
        _redirect(idxb, off, nh_u)
        if gather:
            for cp in cps2:
                cp.wait()
        for j in range(SUB):
            pltpu.async_copy(slot(j), table.at[idxb.at[SUB + j]], sem_s,
                             add=True)
        return _

    lax.fori_loop(0, GPT // 2, body, None)

    drain_scatters(idxb)
    pltpu.make_async_copy(eidx_hbm.at[0], idxa, sem_ia).wait()

    plsc.subcore_barrier()

    ob = s * ROWS_PER_TILE
    pltpu.sync_copy(table.at[pl.ds(ob, ROWS_PER_TILE)],
                    out_hbm.at[pl.ds(c * NHALF + ob, ROWS_PER_TILE)])


def _make_agg(gather: bool):
    if gather:
        body = functools.partial(_agg_body, gather=True)
    else:
        def body(eidx_hbm, out_hbm, idxa, idxb, rows, table,
                 sem_ia, sem_ib, sem_g, sem_s):
            _agg_body(None, eidx_hbm, out_hbm, idxa, idxb, rows, table,
                      sem_ia, sem_ib, sem_g, sem_s, gather=False)
    return pl.kernel(
        body,
        out_type=jax.ShapeDtypeStruct((NP, H), jnp.float32),
        mesh=plsc.VectorSubcoreMesh(core_axis_name="c", subcore_axis_name="s"),
        scratch_types=[
            pltpu.VMEM((2 * SUB, 128), jnp.int32),
            pltpu.VMEM((2 * SUB, 128), jnp.int32),
            pltpu.VMEM((SUB * 128, H), jnp.float32),
            pltpu.VMEM_SHARED((TBL_ROWS, H), jnp.float32),
            pltpu.SemaphoreType.DMA,
            pltpu.SemaphoreType.DMA,
            pltpu.SemaphoreType.DMA,
            pltpu.SemaphoreType.DMA,
        ],
        compiler_params=pltpu.CompilerParams(use_tc_tiling_on_sc=False),
    )


def _atom_body(x_ref, emb_ref, deg_ref, w1_ref, hws_ref, hw_ref, dinv_ref):
    xb = x_ref[...]
    h0 = jnp.zeros((R, H), jnp.float32)
    for i in range(9):
        xi = xb[:, i:i + 1]
        oh = (lax.broadcasted_iota(jnp.int32, (R, 128), 1) == xi)
        h0 = h0 + jnp.dot(oh.astype(jnp.float32), emb_ref[i],
                          preferred_element_type=jnp.float32)
    deg = deg_ref[:, 0:1] + 1.0
    dinv = lax.rsqrt(deg)
    hw = jnp.dot(h0, w1_ref[...], preferred_element_type=jnp.float32)
    hws_ref[...] = hw * dinv
    hw_ref[...] = hw
    dinv_ref[...] = jnp.broadcast_to(dinv, (R, 8))


def _gcn_post_body(s_ref, hw_ref, dinv_ref, b_ref, w_ref, hws_o, hw_o, *,
                   scale_next: bool):
    dinv = dinv_ref[:, 0:1]
    h = jax.nn.relu(dinv * s_ref[...] + dinv * dinv * hw_ref[...] + b_ref[...])
    hwn = jnp.dot(h, w_ref[...], preferred_element_type=jnp.float32)
    if scale_next:
        hws_o[...] = hwn * dinv
        hw_o[...] = hwn
    else:
        hws_o[...] = h
        hw_o[...] = hwn


def _gru_body(h_ref, m_ref, wih_ref, whh_ref, bih_ref, bhh_ref, wg_ref,
              h_o, u_o=None, *, last: bool):
    gi = jnp.dot(m_ref[...], wih_ref[...],
                 preferred_element_type=jnp.float32) + bih_ref[...]
    gh = jnp.dot(h_ref[...], whh_ref[...],
                 preferred_element_type=jnp.float32) + bhh_ref[...]
    r = jax.nn.sigmoid(gi[:, 0:H] + gh[:, 0:H])
    z = jax.nn.sigmoid(gi[:, H:2 * H] + gh[:, H:2 * H])
    cand = jnp.tanh(gi[:, 2 * H:3 * H] + r * gh[:, 2 * H:3 * H])
    hn = (1.0 - z) * cand + z * h_ref[...]
    h_o[...] = hn
    if not last:
        u_o[...] = jnp.dot(hn, wg_ref[...], preferred_element_type=jnp.float32)


def _pool_body(h_ref, bid_ref, linw_ref, linb_ref, out_ref, sums, cnt):
    pid = pl.program_id(0)

    @pl.when(pid == 0)
    def _():
        sums[...] = jnp.zeros((G, H), jnp.float32)
        cnt[...] = jnp.zeros((G, 1), jnp.float32)

    mat = (lax.broadcasted_iota(jnp.int32, (G, R), 0) == bid_ref[...])
    matf = mat.astype(jnp.float32)
    sums[...] += jnp.dot(matf, h_ref[...], preferred_element_type=jnp.float32)
    cnt[...] += jnp.sum(matf, axis=1, keepdims=True)

    @pl.when(pid == NP // R - 1)
    def _():
        pooled = sums[...] / jnp.maximum(cnt[...], 1.0)
        out_ref[...] = jnp.dot(pooled, linw_ref[...],
                               preferred_element_type=jnp.float32) + linb_ref[...]


def _row_spec(width):
    return pl.BlockSpec((R, width), lambda i: (i, 0))


def _whole(shape):
    nd = len(shape)
    return pl.BlockSpec(shape, lambda i, _nd=nd: (0,) * _nd)


def _row_out(width):
    return jax.ShapeDtypeStruct((NP, width), jnp.float32)


def kernel(x, edge_index, batch_ids, emb, W1, b1, W2, b2, Wg, Wih, Whh,
           bih, bhh, linW, linb):
    f32 = jnp.float32
    x_p = jnp.zeros((NP, 16), jnp.int32).at[:N, :9].set(x)
    emb_p = jnp.zeros((9, 128, H), f32).at[:, :100, :].set(emb)
    src = edge_index[0]
    dst = edge_index[1]
    pad_e = EPAD - E
    src_r = jnp.concatenate([src, jnp.zeros((pad_e,), jnp.int32)]
                            ).reshape(NGRP, SUB, 128)
    dst_r = jnp.concatenate([dst, jnp.full((pad_e,), NP, jnp.int32)]
                            ).reshape(NGRP, SUB, 128)
    eidx = jnp.concatenate(
        [jnp.concatenate([src_r, dst_r], axis=1),
         jnp.zeros((1, 2 * SUB, 128), jnp.int32)], axis=0)
    bid_p = jnp.full((1, NP), G, jnp.int32).at[0, :N].set(batch_ids)
    b1r = b1.reshape(1, H)
    b2r = b2.reshape(1, H)
    bihr = bih.reshape(1, 3 * H)
    bhhr = bhh.reshape(1, 3 * H)
    WihT = Wih.T
    WhhT = Whh.T
    linbr = linb.reshape(1, C)

    agg = _make_agg(gather=True)
    deg_agg = _make_agg(gather=False)

    deg_tab = deg_agg(eidx)

    grid = (NP // R,)
    hws1, hw1, dinv = pl.pallas_call(
        _atom_body,
        grid=grid,
        in_specs=[_row_spec(16), _whole((9, 128, H)), _row_spec(H), _whole((H, H))],
        out_specs=[_row_spec(H), _row_spec(H), _row_spec(8)],
        out_shape=[_row_out(H), _row_out(H), _row_out(8)],
    )(x_p, emb_p, deg_tab, W1)

    s1 = agg(hws1, eidx)

    hws2, hw2 = pl.pallas_call(
        functools.partial(_gcn_post_body, scale_next=True),
        grid=grid,
        in_specs=[_row_spec(H), _row_spec(H), _row_spec(8), _whole((1, H)),
                  _whole((H, H))],
        out_specs=[_row_spec(H), _row_spec(H)],
        out_shape=[_row_out(H), _row_out(H)],
    )(s1, hw1, dinv, b1r, W2)

    s2 = agg(hws2, eidx)

    h, u = pl.pallas_call(
        functools.partial(_gcn_post_body, scale_next=False),
        grid=grid,
        in_specs=[_row_spec(H), _row_spec(H), _row_spec(8), _whole((1, H)),
                  _whole((H, H))],
        out_specs=[_row_spec(H), _row_spec(H)],
        out_shape=[_row_out(H), _row_out(H)],
    )(s2, hw2, dinv, b2r, Wg[0])

    for i in range(5):
        m = agg(u, eidx)
        last = i == 4
        outs = pl.pallas_call(
            functools.partial(_gru_body, last=last),
            grid=grid,
            in_specs=[_row_spec(H), _row_spec(H), _whole((H, 3 * H)),
                      _whole((H, 3 * H)), _whole((1, 3 * H)), _whole((1, 3 * H)),
                      _whole((H, H))],
            out_specs=[_row_spec(H)] if last else [_row_spec(H), _row_spec(H)],
            out_shape=[_row_out(H)] if last else [_row_out(H), _row_out(H)],
        )(h, m, WihT, WhhT, bihr, bhhr, Wg[min(i + 1, 4)])
        if last:
            h = outs[0]
        else:
            h, u = outs

    out = pl.pallas_call(
        _pool_body,
        grid=grid,
        in_specs=[_row_spec(H), pl.BlockSpec((1, R), lambda i: (0, i)),
                  _whole((H, C)), _whole((1, C))],
        out_specs=pl.BlockSpec((G, C), lambda i: (0, 0)),
        out_shape=jax.ShapeDtypeStruct((G, C), f32),
        scratch_shapes=[pltpu.VMEM((G, H), f32), pltpu.VMEM((G, 1), f32)],
    )(h, bid_p, linW, linbr)
    return out


# speedup vs baseline: 1.0116x; 1.0116x over previous
import functools

import jax
import jax.numpy as jnp
from jax import lax
from jax.experimental import pallas as pl
from jax.experimental.pallas import tpu as pltpu
from jax.experimental.pallas import tpu_sc as plsc

N = 100000
E = 1600000
H = 32
C = 10
G = 32

NC = 2
NS = 16
LANES = 16

R = 2048
NP = 49 * R
NHALF = NP // 2
ROWS_PER_TILE = NHALF // NS
NTRASH = 1024
TBL_ROWS = NHALF + NTRASH
ZSTRIPE = TBL_ROWS // NS

SUB = 4
GRP = SUB * 128
GPT = 196
EPT = GPT * GRP
EPAD = NS * EPT
NGRP = NS * GPT


def _fill_rows(rows, value, n):
    fill = jnp.full((LANES,), value, jnp.float32)

    def fill_body(i, _):
        rows[i, pl.ds(0, LANES)] = fill
        rows[i, pl.ds(LANES, LANES)] = fill
        return _

    lax.fori_loop(0, n, fill_body, None)


def _redirect(idx, off, nh_u):
    for j in range(SUB):
        for l in range(8):
            v = idx[SUB + j, pl.ds(l * LANES, LANES)]
            u = lax.bitcast_convert_type(v - off, jnp.uint32)
            trash = jnp.uint32(NHALF) + (u & jnp.uint32(NTRASH - 1))
            u = jnp.where(u < nh_u, u, trash)
            idx[SUB + j, pl.ds(l * LANES, LANES)] = lax.bitcast_convert_type(
                u, jnp.int32)


def _agg_body(in_hbm, eidx_hbm, out_hbm, idxa, idxb, rows, table,
              sem_ia, sem_ib, sem_g, sem_s, *, gather: bool):
    c = lax.axis_index("c")
    s = lax.axis_index("s")
    nrows = SUB * 128

    _fill_rows(rows, 0.0, nrows)
    zb = s * ZSTRIPE
    nfull = ZSTRIPE // nrows

    def zero_body(i, _):
        pltpu.sync_copy(rows, table.at[pl.ds(zb + i * nrows, nrows)])
        return _

    lax.fori_loop(0, nfull, zero_body, None)
    rem = ZSTRIPE - nfull * nrows
    pltpu.sync_copy(rows.at[pl.ds(0, rem)],
                    table.at[pl.ds(zb + nfull * nrows, rem)])
    if not gather:
        _fill_rows(rows, 1.0, nrows)

    plsc.subcore_barrier()

    nh_u = jnp.uint32(NHALF)
    off = (c * NHALF).astype(jnp.int32)
    base = s * GPT

    def slot(j):
        return rows.at[pl.ds(j * 128, 128)]

    def drain_scatters(idx):
        for j in range(SUB):
            pltpu.make_async_copy(slot(j), table.at[idx.at[SUB + j]],
                                  sem_s).wait()

    pltpu.async_copy(eidx_hbm.at[base], idxa, sem_ia)

    def body(i, _):
        g = base + 2 * i

        pltpu.make_async_copy(eidx_hbm.at[0], idxa, sem_ia).wait()

        @pl.when(i > 0)
        def _():
            drain_scatters(idxb)
        if gather:
            cps = [pltpu.async_copy(in_hbm.at[idxa.at[j]], slot(j), sem_g)
                   for j in range(SUB)]
        pltpu.async_copy(eidx_hbm.at[g + 1], idxb, sem_ib)
        _redirect(idxa, off, nh_u)
        if gather:
            for cp in cps:
                cp.wait()
        for j in range(SUB):
            pltpu.async_copy(slot(j), table.at[idxa.at[SUB + j]], sem_s,
                             add=True)

        pltpu.make_async_copy(eidx_hbm.at[0], idxb, sem_ib).wait()
        drain_scatters(idxa)
        if gather:
            cps2 = [pltpu.async_copy(in_hbm.at[idxb.at[j]], slot(j), sem_g)
                    for j in range(SUB)]
        pltpu.async_copy(eidx_hbm.at[g + 2], idxa, sem_ia)
        _redirect(idxb, off, nh_u)
        if gather:
            for cp in cps2:
                cp.wait()
        for j in range(SUB):
            pltpu.async_copy(slot(j), table.at[idxb.at[SUB + j]], sem_s,
                             add=True)
        return _

    lax.fori_loop(0, GPT // 2, body, None)

    drain_scatters(idxb)
    pltpu.make_async_copy(eidx_hbm.at[0], idxa, sem_ia).wait()

    plsc.subcore_barrier()

    ob = s * ROWS_PER_TILE
    pltpu.sync_copy(table.at[pl.ds(ob, ROWS_PER_TILE)],
                    out_hbm.at[pl.ds(c * NHALF + ob, ROWS_PER_TILE)])


def _make_agg(gather: bool):
    if gather:
        body = functools.partial(_agg_body, gather=True)
    else:
        def body(eidx_hbm, out_hbm, idxa, idxb, rows, table,
                 sem_ia, sem_ib, sem_g, sem_s):
            _agg_body(None, eidx_hbm, out_hbm, idxa, idxb, rows, table,
                      sem_ia, sem_ib, sem_g, sem_s, gather=False)
    return pl.kernel(
        body,
        out_type=jax.ShapeDtypeStruct((NP, H), jnp.float32),
        mesh=plsc.VectorSubcoreMesh(core_axis_name="c", subcore_axis_name="s"),
        scratch_types=[
            pltpu.VMEM((2 * SUB, 128), jnp.int32),
            pltpu.VMEM((2 * SUB, 128), jnp.int32),
            pltpu.VMEM((SUB * 128, H), jnp.float32),
            pltpu.VMEM_SHARED((TBL_ROWS, H), jnp.float32),
            pltpu.SemaphoreType.DMA,
            pltpu.SemaphoreType.DMA,
            pltpu.SemaphoreType.DMA,
            pltpu.SemaphoreType.DMA,
        ],
        compiler_params=pltpu.CompilerParams(use_tc_tiling_on_sc=False),
    )


def _atom_body(x_ref, emb_ref, deg_ref, w1_ref, hws_ref, hw_ref, dinv_ref):
    xb = x_ref[...]
    ohs = []
    for i in range(9):
        xi = xb[:, i:i + 1]
        oh = (lax.broadcasted_iota(jnp.int32, (R, 128), 1) == xi)
        ohs.append(oh.astype(jnp.float32))
    h0 = jnp.dot(jnp.concatenate(ohs, axis=1), emb_ref[...],
                 preferred_element_type=jnp.float32)
    deg = deg_ref[:, 0:1] + 1.0
    dinv = lax.rsqrt(deg)
    hw = jnp.dot(h0, w1_ref[...], preferred_element_type=jnp.float32)
    hws_ref[...] = hw * dinv
    hw_ref[...] = hw
    dinv_ref[...] = jnp.broadcast_to(dinv, (R, 8))


def _gcn_post_body(s_ref, hw_ref, dinv_ref, b_ref, w_ref, hws_o, hw_o, *,
                   scale_next: bool):
    dinv = dinv_ref[:, 0:1]
    h = jax.nn.relu(dinv * s_ref[...] + dinv * dinv * hw_ref[...] + b_ref[...])
    hwn = jnp.dot(h, w_ref[...], preferred_element_type=jnp.float32)
    if scale_next:
        hws_o[...] = hwn * dinv
        hw_o[...] = hwn
    else:
        hws_o[...] = h
        hw_o[...] = hwn


def _gru_body(h_ref, m_ref, wr_ref, wz_ref, wn_ref, ur_ref, uz_ref, un_ref,
              brz_ref, bin_ref, bhn_ref, wg_ref, h_o, u_o=None, *, last: bool):
    m = m_ref[...]
    h = h_ref[...]
    f32 = jnp.float32
    r = jax.nn.sigmoid(jnp.dot(m, wr_ref[...], preferred_element_type=f32)
                       + jnp.dot(h, ur_ref[...], preferred_element_type=f32)
                       + brz_ref[:, 0:H])
    z = jax.nn.sigmoid(jnp.dot(m, wz_ref[...], preferred_element_type=f32)
                       + jnp.dot(h, uz_ref[...], preferred_element_type=f32)
                       + brz_ref[:, H:2 * H])
    ih = jnp.dot(m, wn_ref[...], preferred_element_type=f32) + bin_ref[...]
    hn_ = jnp.dot(h, un_ref[...], preferred_element_type=f32) + bhn_ref[...]
    cand = jnp.tanh(ih + r * hn_)
    hn = (1.0 - z) * cand + z * h
    h_o[...] = hn
    if not last:
        u_o[...] = jnp.dot(hn, wg_ref[...], preferred_element_type=f32)


def _pool_body(h_ref, bid_ref, linw_ref, linb_ref, out_ref, sums, cnt):
    pid = pl.program_id(0)

    @pl.when(pid == 0)
    def _():
        sums[...] = jnp.zeros((G, H), jnp.float32)
        cnt[...] = jnp.zeros((G, 1), jnp.float32)

    mat = (lax.broadcasted_iota(jnp.int32, (G, R), 0) == bid_ref[...])
    matf = mat.astype(jnp.float32)
    sums[...] += jnp.dot(matf, h_ref[...], preferred_element_type=jnp.float32)
    cnt[...] += jnp.sum(matf, axis=1, keepdims=True)

    @pl.when(pid == NP // R - 1)
    def _():
        pooled = sums[...] / jnp.maximum(cnt[...], 1.0)
        out_ref[...] = jnp.dot(pooled, linw_ref[...],
                               preferred_element_type=jnp.float32) + linb_ref[...]


def _row_spec(width):
    return pl.BlockSpec((R, width), lambda i: (i, 0))


def _whole(shape):
    nd = len(shape)
    return pl.BlockSpec(shape, lambda i, _nd=nd: (0,) * _nd)


def _row_out(width):
    return jax.ShapeDtypeStruct((NP, width), jnp.float32)


def kernel(x, edge_index, batch_ids, emb, W1, b1, W2, b2, Wg, Wih, Whh,
           bih, bhh, linW, linb):
    f32 = jnp.float32
    x_p = jnp.zeros((NP, 16), jnp.int32).at[:N, :9].set(x)
    emb_p = jnp.zeros((9, 128, H), f32).at[:, :100, :].set(emb).reshape(9 * 128, H)
    src = edge_index[0]
    dst = edge_index[1]
    pad_e = EPAD - E
    src_r = jnp.concatenate([src, jnp.zeros((pad_e,), jnp.int32)]
                            ).reshape(NGRP, SUB, 128)
    dst_r = jnp.concatenate([dst, jnp.full((pad_e,), NP, jnp.int32)]
                            ).reshape(NGRP, SUB, 128)
    eidx = jnp.concatenate(
        [jnp.concatenate([src_r, dst_r], axis=1),
         jnp.zeros((1, 2 * SUB, 128), jnp.int32)], axis=0)
    bid_p = jnp.full((1, NP), G, jnp.int32).at[0, :N].set(batch_ids)
    b1r = b1.reshape(1, H)
    b2r = b2.reshape(1, H)
    WihT = Wih.T
    WhhT = Whh.T
    wr, wz, wn = WihT[:, 0:H], WihT[:, H:2 * H], WihT[:, 2 * H:3 * H]
    ur, uz, un = WhhT[:, 0:H], WhhT[:, H:2 * H], WhhT[:, 2 * H:3 * H]
    brz = (bih[0:2 * H] + bhh[0:2 * H]).reshape(1, 2 * H)
    binr = bih[2 * H:3 * H].reshape(1, H)
    bhnr = bhh[2 * H:3 * H].reshape(1, H)
    linbr = linb.reshape(1, C)

    agg = _make_agg(gather=True)
    deg_agg = _make_agg(gather=False)

    deg_tab = deg_agg(eidx)

    grid = (NP // R,)
    hws1, hw1, dinv = pl.pallas_call(
        _atom_body,
        grid=grid,
        in_specs=[_row_spec(16), _whole((9 * 128, H)), _row_spec(H), _whole((H, H))],
        out_specs=[_row_spec(H), _row_spec(H), _row_spec(8)],
        out_shape=[_row_out(H), _row_out(H), _row_out(8)],
    )(x_p, emb_p, deg_tab, W1)

    s1 = agg(hws1, eidx)

    hws2, hw2 = pl.pallas_call(
        functools.partial(_gcn_post_body, scale_next=True),
        grid=grid,
        in_specs=[_row_spec(H), _row_spec(H), _row_spec(8), _whole((1, H)),
                  _whole((H, H))],
        out_specs=[_row_spec(H), _row_spec(H)],
        out_shape=[_row_out(H), _row_out(H)],
    )(s1, hw1, dinv, b1r, W2)

    s2 = agg(hws2, eidx)

    h, u = pl.pallas_call(
        functools.partial(_gcn_post_body, scale_next=False),
        grid=grid,
        in_specs=[_row_spec(H), _row_spec(H), _row_spec(8), _whole((1, H)),
                  _whole((H, H))],
        out_specs=[_row_spec(H), _row_spec(H)],
        out_shape=[_row_out(H), _row_out(H)],
    )(s2, hw2, dinv, b2r, Wg[0])

    for i in range(5):
        m = agg(u, eidx)
        last = i == 4
        outs = pl.pallas_call(
            functools.partial(_gru_body, last=last),
            grid=grid,
            in_specs=[_row_spec(H), _row_spec(H)]
                     + [_whole((H, H))] * 6
                     + [_whole((1, 2 * H)), _whole((1, H)), _whole((1, H)),
                        _whole((H, H))],
            out_specs=[_row_spec(H)] if last else [_row_spec(H), _row_spec(H)],
            out_shape=[_row_out(H)] if last else [_row_out(H), _row_out(H)],
        )(h, m, wr, wz, wn, ur, uz, un, brz, binr, bhnr, Wg[min(i + 1, 4)])
        if last:
            h = outs[0]
        else:
            h, u = outs

    out = pl.pallas_call(
        _pool_body,
        grid=grid,
        in_specs=[_row_spec(H), pl.BlockSpec((1, R), lambda i: (0, i)),
                  _whole((H, C)), _whole((1, C))],
        out_specs=pl.BlockSpec((G, C), lambda i: (0, 0)),
        out_shape=jax.ShapeDtypeStruct((G, C), f32),
        scratch_shapes=[pltpu.VMEM((G, H), f32), pltpu.VMEM((G, 1), f32)],
    )(h, bid_p, linW, linbr)
    return out
